# baseline (speedup 1.0000x reference)
import jax
import jax.numpy as jnp
from jax.experimental import pallas as pl
from jax.experimental.pallas import tpu as pltpu

_B, _D, _H, _L = 16384, 32, 64, 100


def _fused_kernel(x_ref, t_ref, w1_ref, b1_ref, g1_ref, beta1_ref, rm1_ref,
                  rv1_ref, w2_ref, b2_ref, g2_ref, beta2_ref, rm2_ref,
                  rv2_ref, w3_ref, b3_ref, out_ref):
    eps = 1e-5

    s1 = g1_ref[...] * jax.lax.rsqrt(rv1_ref[...] + eps)
    c1 = (b1_ref[...] - rm1_ref[...]) * s1 + beta1_ref[...]
    s2 = g2_ref[...] * jax.lax.rsqrt(rv2_ref[...] + eps)
    c2 = (b2_ref[...] - rm2_ref[...]) * s2 + beta2_ref[...]

    x = x_ref[...]
    h = jnp.maximum(
        jnp.dot(x, w1_ref[...] * s1, preferred_element_type=jnp.float32)
        + c1, 0.0)
    h = jnp.maximum(
        jnp.dot(h, w2_ref[...] * s2, preferred_element_type=jnp.float32)
        + c2, 0.0)
    logits = (jnp.dot(h, w3_ref[...], preferred_element_type=jnp.float32)
              + b3_ref[...])

    m = jnp.max(logits, axis=1, keepdims=True)
    lse = m + jnp.log(jnp.sum(jnp.exp(logits - m), axis=1, keepdims=True))

    lane = jax.lax.broadcasted_iota(jnp.int32, (_B, _L), 1)
    onehot = lane == t_ref[...].reshape(_B, 1)
    part = jnp.sum(jnp.where(onehot, logits, 0.0)) - jnp.sum(lse)
    out_ref[0, 0] = part * (-1.0 / _B)


def kernel(x_start, target, W1, b1, g1, beta1, rm1, rv1, W2, b2, g2, beta2,
           rm2, rv2, W3, b3):
    tgt = target.astype(jnp.int32)
    row = lambda v: v.reshape(1, -1)

    loss = pl.pallas_call(
        _fused_kernel,
        out_specs=pl.BlockSpec(memory_space=pltpu.SMEM),
        out_shape=jax.ShapeDtypeStruct((1, 1), jnp.float32),
    )(x_start, tgt, W1, row(b1), row(g1), row(beta1), row(rm1), row(rv1),
      W2, row(b2), row(g2), row(beta2), row(rm2), row(rv2), W3, row(b3))
    return loss[0, 0]


# speedup vs baseline: 1.0345x; 1.0345x over previous
import jax
import jax.numpy as jnp
from jax.experimental import pallas as pl
from jax.experimental.pallas import tpu as pltpu

_B, _D, _H, _L = 16384, 32, 64, 100
_LP = 128
_BM = 8192
_GRID = _B // _BM


def _fused_kernel(x_ref, t_ref, w1_ref, b1_ref, g1_ref, beta1_ref, rm1_ref,
                  rv1_ref, w2_ref, b2_ref, g2_ref, beta2_ref, rm2_ref,
                  rv2_ref, w3_ref, b3_ref, out_ref):
    i = pl.program_id(0)
    eps = 1e-5

    s1 = g1_ref[...] * jax.lax.rsqrt(rv1_ref[...] + eps)
    c1 = (b1_ref[...] - rm1_ref[...]) * s1 + beta1_ref[...]
    s2 = g2_ref[...] * jax.lax.rsqrt(rv2_ref[...] + eps)
    c2 = (b2_ref[...] - rm2_ref[...]) * s2 + beta2_ref[...]

    x = x_ref[...]
    h = jnp.maximum(
        jnp.dot(x, w1_ref[...] * s1, preferred_element_type=jnp.float32)
        + c1, 0.0)
    h = jnp.maximum(
        jnp.dot(h, w2_ref[...] * s2, preferred_element_type=jnp.float32)
        + c2, 0.0)
    w3 = jnp.pad(w3_ref[...], ((0, 0), (0, _LP - _L)))
    b3 = jnp.pad(b3_ref[...], ((0, 0), (0, _LP - _L)),
                 constant_values=-1e30)
    logits = (jnp.dot(h, w3, preferred_element_type=jnp.float32)
              + b3)

    m = jnp.max(logits, axis=1, keepdims=True)
    lse = m + jnp.log(jnp.sum(jnp.exp(logits - m), axis=1, keepdims=True))

    lane = jax.lax.broadcasted_iota(jnp.int32, (1, _LP), 1)
    onehot = lane == t_ref[...].reshape(_BM, 1)
    part = (jnp.sum(jnp.where(onehot, logits, 0.0)) - jnp.sum(lse))

    @pl.when(i == 0)
    def _():
        out_ref[0, 0] = 0.0

    out_ref[0, 0] += part

    @pl.when(i == _GRID - 1)
    def _():
        out_ref[0, 0] = out_ref[0, 0] * (-1.0 / _B)


def kernel(x_start, target, W1, b1, g1, beta1, rm1, rv1, W2, b2, g2, beta2,
           rm2, rv2, W3, b3):
    tgt = target.astype(jnp.int32)
    row = lambda v: v.reshape(1, -1)

    full = lambda shape: pl.BlockSpec(shape, lambda i: tuple(0 for _ in shape))
    loss = pl.pallas_call(
        _fused_kernel,
        grid=(_GRID,),
        in_specs=[
            pl.BlockSpec((_BM, _D), lambda i: (i, 0)),
            pl.BlockSpec((_BM,), lambda i: (i,)),
            full((_D, _H)),
            full((1, _H)), full((1, _H)), full((1, _H)),
            full((1, _H)), full((1, _H)),
            full((_H, _H)),
            full((1, _H)), full((1, _H)), full((1, _H)),
            full((1, _H)), full((1, _H)),
            full((_H, _L)),
            full((1, _L)),
        ],
        out_specs=pl.BlockSpec(memory_space=pltpu.SMEM),
        out_shape=jax.ShapeDtypeStruct((1, 1), jnp.float32),
    )(x_start, tgt, W1, row(b1), row(g1), row(beta1), row(rm1), row(rv1),
      W2, row(b2), row(g2), row(beta2), row(rm2), row(rv2), W3, row(b3))
    return loss[0, 0]


# speedup vs baseline: 1.1383x; 1.1003x over previous
import jax
import jax.numpy as jnp
from jax.experimental import pallas as pl
from jax.experimental.pallas import tpu as pltpu

_B, _D, _H, _L = 16384, 32, 64, 100
_LP = 128
_BM = 8192
_GRID = _B // _BM

_DN0 = (((0,), (0,)), ((), ()))


def _fused_kernel(x_ref, t_ref, w1_ref, b1_ref, g1_ref, beta1_ref, rm1_ref,
                  rv1_ref, w2_ref, b2_ref, g2_ref, beta2_ref, rm2_ref,
                  rv2_ref, w3_ref, b3_ref, out_ref):
    i = pl.program_id(0)
    eps = 1e-5

    col = lambda r: r[...].reshape(_H, 1)
    s1 = col(g1_ref) * jax.lax.rsqrt(col(rv1_ref) + eps)
    c1 = (col(b1_ref) - col(rm1_ref)) * s1 + col(beta1_ref)
    s2 = col(g2_ref) * jax.lax.rsqrt(col(rv2_ref) + eps)
    c2 = (col(b2_ref) - col(rm2_ref)) * s2 + col(beta2_ref)

    w1s = w1_ref[...] * s1.reshape(1, _H)
    w2s = w2_ref[...] * s2.reshape(1, _H)

    xT_dot = lambda a, b: jax.lax.dot_general(
        a, b, dimension_numbers=_DN0, preferred_element_type=jnp.float32)
    h = jnp.maximum(
        jax.lax.dot_general(w1s, x_ref[...],
                            dimension_numbers=(((0,), (1,)), ((), ())),
                            preferred_element_type=jnp.float32) + c1, 0.0)
    h = jnp.maximum(xT_dot(w2s, h) + c2, 0.0)

    w3p = jnp.pad(w3_ref[...], ((0, 0), (0, _LP - _L)))
    b3p = jnp.pad(b3_ref[...], ((0, 0), (0, _LP - _L)),
                  constant_values=-1e30).reshape(_LP, 1)
    logits = xT_dot(w3p, h) + b3p

    m = jnp.max(logits, axis=0, keepdims=True)
    s = jnp.sum(jnp.exp(logits - m), axis=0, keepdims=True)
    sum_lse = jnp.sum(m + jnp.log(s))

    sub = jax.lax.broadcasted_iota(jnp.int32, (_LP, 1), 0)
    onehot = sub == t_ref[...].reshape(1, _BM)
    part = jnp.sum(jnp.where(onehot, logits, 0.0)) - sum_lse

    @pl.when(i == 0)
    def _():
        out_ref[0, 0] = 0.0

    out_ref[0, 0] += part

    @pl.when(i == _GRID - 1)
    def _():
        out_ref[0, 0] = out_ref[0, 0] * (-1.0 / _B)


def kernel(x_start, target, W1, b1, g1, beta1, rm1, rv1, W2, b2, g2, beta2,
           rm2, rv2, W3, b3):
    tgt = target.astype(jnp.int32)
    row = lambda v: v.reshape(1, -1)

    full = lambda shape: pl.BlockSpec(shape, lambda i: tuple(0 for _ in shape))
    loss = pl.pallas_call(
        _fused_kernel,
        grid=(_GRID,),
        in_specs=[
            pl.BlockSpec((_BM, _D), lambda i: (i, 0)),
            pl.BlockSpec((_BM,), lambda i: (i,)),
            full((_D, _H)),
            full((1, _H)), full((1, _H)), full((1, _H)),
            full((1, _H)), full((1, _H)),
            full((_H, _H)),
            full((1, _H)), full((1, _H)), full((1, _H)),
            full((1, _H)), full((1, _H)),
            full((_H, _L)),
            full((1, _L)),
        ],
        out_specs=pl.BlockSpec(memory_space=pltpu.SMEM),
        out_shape=jax.ShapeDtypeStruct((1, 1), jnp.float32),
    )(x_start, tgt, W1, row(b1), row(g1), row(beta1), row(rm1), row(rv1),
      W2, row(b2), row(g2), row(beta2), row(rm2), row(rv2), W3, row(b3))
    return loss[0, 0]


# speedup vs baseline: 2.5134x; 2.2081x over previous
import jax
import jax.numpy as jnp
from jax.experimental import pallas as pl
from jax.experimental.pallas import tpu as pltpu

_B, _D, _H, _L = 16384, 32, 64, 100
_LP = 104
_BM = 16384
_GRID = _B // _BM

_DN0 = (((0,), (0,)), ((), ()))


def _fused_kernel(x_ref, t_ref, w1_ref, b1_ref, g1_ref, beta1_ref, rm1_ref,
                  rv1_ref, w2_ref, b2_ref, g2_ref, beta2_ref, rm2_ref,
                  rv2_ref, w3_ref, b3_ref, out_ref):
    i = pl.program_id(0)
    eps = 1e-5

    col = lambda r: r[...].reshape(_H, 1)
    s1 = col(g1_ref) * jax.lax.rsqrt(col(rv1_ref) + eps)
    c1 = (col(b1_ref) - col(rm1_ref)) * s1 + col(beta1_ref)
    s2 = col(g2_ref) * jax.lax.rsqrt(col(rv2_ref) + eps)
    c2 = (col(b2_ref) - col(rm2_ref)) * s2 + col(beta2_ref)

    w1s = w1_ref[...] * s1.reshape(1, _H)
    w2s = w2_ref[...] * s2.reshape(1, _H)

    xT_dot = lambda a, b: jax.lax.dot_general(
        a, b, dimension_numbers=_DN0, preferred_element_type=jnp.float32)
    h = jnp.maximum(xT_dot(w1s, x_ref[...]) + c1, 0.0)
    h = jnp.maximum(xT_dot(w2s, h) + c2, 0.0)

    w3p = jnp.pad(w3_ref[...], ((0, 0), (0, _LP - _L)))
    b3p = jnp.pad(b3_ref[...], ((0, 0), (0, _LP - _L)),
                  constant_values=-1e30).reshape(_LP, 1)
    logits = xT_dot(w3p, h) + b3p

    m = jnp.max(logits, axis=0, keepdims=True)
    s = jnp.sum(jnp.exp(logits - m), axis=0, keepdims=True)
    sum_lse = jnp.sum(m + jnp.log(s))

    sub = jax.lax.broadcasted_iota(jnp.int32, (_LP, 1), 0)
    onehot = sub == t_ref[...].reshape(1, _BM)
    part = jnp.sum(jnp.where(onehot, logits, 0.0)) - sum_lse

    @pl.when(i == 0)
    def _():
        out_ref[0, 0] = 0.0

    out_ref[0, 0] += part

    @pl.when(i == _GRID - 1)
    def _():
        out_ref[0, 0] = out_ref[0, 0] * (-1.0 / _B)


def kernel(x_start, target, W1, b1, g1, beta1, rm1, rv1, W2, b2, g2, beta2,
           rm2, rv2, W3, b3):
    tgt = target.astype(jnp.int32)
    xt = x_start.T
    row = lambda v: v.reshape(1, -1)

    full = lambda shape: pl.BlockSpec(shape, lambda i: tuple(0 for _ in shape))
    loss = pl.pallas_call(
        _fused_kernel,
        grid=(_GRID,),
        in_specs=[
            pl.BlockSpec((_D, _BM), lambda i: (0, i)),
            pl.BlockSpec((_BM,), lambda i: (i,)),
            full((_D, _H)),
            full((1, _H)), full((1, _H)), full((1, _H)),
            full((1, _H)), full((1, _H)),
            full((_H, _H)),
            full((1, _H)), full((1, _H)), full((1, _H)),
            full((1, _H)), full((1, _H)),
            full((_H, _L)),
            full((1, _L)),
        ],
        out_specs=pl.BlockSpec(memory_space=pltpu.SMEM),
        out_shape=jax.ShapeDtypeStruct((1, 1), jnp.float32),
    )(xt, tgt, W1, row(b1), row(g1), row(beta1), row(rm1), row(rv1),
      W2, row(b2), row(g2), row(beta2), row(rm2), row(rv2), W3, row(b3))
    return loss[0, 0]
